# baseline (speedup 1.0000x reference)
import functools

import jax
import jax.numpy as jnp
from jax import lax
from jax.experimental import pallas as pl
from jax.experimental.pallas import tpu as pltpu
from jax.experimental.pallas import tpu_sc as plsc

_SEQ = 16384
_T = 50
_B = _SEQ * _T
_D = 64
_NC = 2
_NS = 16
_NW = _NC * _NS
_BPW = _B // _NW
_CH = 200
_SCH = _CH // _T
_NCH = _BPW // _CH
_NSLOT = 8
_NROUNDS = _NCH // _NSLOT


def _embedding_gather(idx, table):
    mesh = plsc.VectorSubcoreMesh(core_axis_name="c", subcore_axis_name="s")

    @functools.partial(
        pl.kernel,
        mesh=mesh,
        compiler_params=pltpu.CompilerParams(use_tc_tiling_on_sc=False),
        out_type=jax.ShapeDtypeStruct((_SEQ, _T, _D), jnp.float32),
        scratch_types=[
            pltpu.VMEM((_BPW,), jnp.int32),
            pltpu.VMEM((_NSLOT, _CH, _D), jnp.float32),
            pltpu.SemaphoreType.DMA((_NSLOT,)),
            pltpu.SemaphoreType.DMA((_NSLOT,)),
        ],
    )
    def k(idx_hbm, table_hbm, out_hbm, idx_v, bufs, gsem, wsem):
        wid = lax.axis_index("s") * _NC + lax.axis_index("c")
        base = wid * _BPW
        seq_base = wid * (_BPW // _T)
        pltpu.sync_copy(idx_hbm.at[pl.ds(base, _BPW)], idx_v)

        def gather_desc(c, b):
            return pltpu.make_async_copy(
                table_hbm.at[idx_v.at[pl.ds(c * _CH, _CH)]],
                bufs.at[b],
                gsem.at[b],
            )

        def wb_desc(c, b, s):
            return pltpu.make_async_copy(
                bufs.at[b].at[pl.ds(s * _T, _T)],
                out_hbm.at[seq_base + c * _SCH + s],
                wsem.at[b],
            )

        def wb_start(c, b):
            for s in range(_SCH):
                wb_desc(c, b, s).start()

        def wb_wait(c, b):
            for s in range(_SCH):
                wb_desc(c, b, s).wait()

        for b in range(_NSLOT):
            gather_desc(b, b).start()

        @pl.loop(0, _NROUNDS)
        def _round(g):
            c0 = g * _NSLOT
            for b in range(_NSLOT):
                gather_desc(c0 + b, b).wait()
                wb_start(c0 + b, b)

            @pl.when(g < _NROUNDS - 1)
            def _prefetch():
                for b in range(_NSLOT):
                    wb_wait(c0 + b, b)
                    gather_desc(c0 + _NSLOT + b, b).start()

        cl = (_NROUNDS - 1) * _NSLOT
        for b in range(_NSLOT):
            wb_wait(cl + b, b)

    return k(idx, table)


def kernel(tokens, table):
    idx = tokens.reshape(-1).astype(jnp.int32)
    return _embedding_gather(idx, table)
